# baseline (speedup 1.0000x reference)
import functools

import jax
import jax.numpy as jnp
from jax import lax
from jax.experimental import pallas as pl
from jax.experimental.pallas import tpu as pltpu
from jax.experimental.pallas import tpu_sc as plsc

NUM_K = 8192
DIM = 256
ALPHA = 0.25
ENT_W = 0.01
BN = 256
CW = 512
LANES = 128


def _vq_tc_body(lat_ref, ln_ref, pn_ref, proto_ref, idx_ref, loss_ref,
                d_scr, e_scr, colacc, sums, *, nsteps, n_total):
    i = pl.program_id(0)

    @pl.when(i == 0)
    def _init():
        colacc[...] = jnp.zeros_like(colacc)
        sums[0, 0] = 0.0

    lt2 = lat_ref[...] + lat_ref[...]
    ln = ln_ref[...]
    nchunks = NUM_K // CW
    nh = CW // LANES
    lanef = lax.broadcasted_iota(jnp.int32, (1, LANES), 1).astype(jnp.float32)

    def body_a(c, carry):
        cmin, cidx = carry
        sl = pl.ds(c * CW, CW)
        ptc = proto_ref[sl, :]
        mmc = lax.dot_general(lt2, ptc, (((1,), (1,)), ((), ())),
                              preferred_element_type=jnp.float32)
        dchunk = (ln + pn_ref[:, sl]) - mmc
        d_scr[:, sl] = dchunk
        hmin = dchunk[:, 0:LANES]
        hcol = jnp.zeros((BN, LANES), jnp.float32)
        for h in range(1, nh):
            half = dchunk[:, h * LANES:(h + 1) * LANES]
            upd = half < hmin
            hmin = jnp.minimum(hmin, half)
            hcol = jnp.where(upd, float(h), hcol)
        updc = hmin < cmin
        cmin = jnp.minimum(cmin, hmin)
        cidx = jnp.where(updc, hcol + (c * nh).astype(jnp.float32), cidx)
        return cmin, cidx

    cmin0 = jnp.full((BN, LANES), jnp.inf, jnp.float32)
    cidx0 = jnp.zeros((BN, LANES), jnp.float32)
    cmin, cidx = lax.fori_loop(0, nchunks, body_a, (cmin0, cidx0))

    minv = jnp.min(cmin, axis=1, keepdims=True)
    j_lane = cidx * float(LANES) + lanef
    j_sel = jnp.where(cmin == minv, j_lane, float(NUM_K))
    idxf = jnp.min(j_sel, axis=1, keepdims=True)
    idx_ref[...] = idxf.astype(jnp.int32)
    sums[0, 0] += jnp.sum(minv)

    def body_b(c, zacc):
        sl = pl.ds(c * CW, CW)
        e = jnp.exp(minv - d_scr[:, sl])
        e_scr[:, sl] = e
        zh = e[:, 0:LANES]
        for h in range(1, nh):
            zh = zh + e[:, h * LANES:(h + 1) * LANES]
        return zacc + zh

    zacc = lax.fori_loop(0, nchunks, body_b,
                         jnp.zeros((BN, LANES), jnp.float32))
    z = jnp.sum(zacc, axis=1, keepdims=True)
    zinv = 1.0 / z

    def body_c(c, carry):
        sl = pl.ds(c * CW, CW)
        scaled = e_scr[:, sl] * zinv
        colacc[:, sl] += jnp.sum(scaled, axis=0, keepdims=True)
        return carry

    lax.fori_loop(0, nchunks, body_c, 0)

    @pl.when(i == nsteps - 1)
    def _fin():
        s = colacc[...] * (1.0 / n_total) + 1e-6
        s = s * (1.0 / jnp.sum(s))
        ent = -jnp.sum(s * jnp.log(s))
        val = (sums[0, 0] * ((1.0 + ALPHA) / (n_total * DIM)) + ENT_W * ent)
        loss_ref[...] = jnp.reshape(val, (1, 1))


def _tc_call(latents, ln, pn, prototypes, interpret=False):
    n = latents.shape[0]
    nsteps = n // BN
    return pl.pallas_call(
        functools.partial(_vq_tc_body, nsteps=nsteps, n_total=n),
        grid=(nsteps,),
        in_specs=[
            pl.BlockSpec((BN, DIM), lambda i: (i, 0)),
            pl.BlockSpec((BN, 1), lambda i: (i, 0)),
            pl.BlockSpec((1, NUM_K), lambda i: (0, 0)),
            pl.BlockSpec((NUM_K, DIM), lambda i: (0, 0)),
        ],
        out_specs=[
            pl.BlockSpec((BN, 1), lambda i: (i, 0)),
            pl.BlockSpec((1, 1), lambda i: (0, 0)),
        ],
        out_shape=[
            jax.ShapeDtypeStruct((n, 1), jnp.int32),
            jax.ShapeDtypeStruct((1, 1), jnp.float32),
        ],
        scratch_shapes=[
            pltpu.VMEM((BN, NUM_K), jnp.float32),
            pltpu.VMEM((BN, NUM_K), jnp.float32),
            pltpu.VMEM((1, NUM_K), jnp.float32),
            pltpu.SMEM((1, 1), jnp.float32),
        ],
        interpret=interpret,
    )(latents, ln, pn, prototypes)


def _sc_gather(table, idx):
    n = idx.shape[0]
    info = plsc.get_sparse_core_info()
    nw = info.num_cores * info.num_subcores
    bpw = n // nw
    ch = 128
    mesh = plsc.VectorSubcoreMesh(core_axis_name="c", subcore_axis_name="s")

    @functools.partial(
        pl.kernel, mesh=mesh,
        out_type=jax.ShapeDtypeStruct((n, DIM), jnp.float32),
        scratch_types=[
            pltpu.VMEM((ch,), jnp.int32),
            pltpu.VMEM((ch, DIM), jnp.float32),
            pltpu.SemaphoreType.DMA,
        ],
    )
    def k(table_hbm, idx_hbm, out_hbm, idx_v, rows_v, sem):
        wid = lax.axis_index("s") * info.num_cores + lax.axis_index("c")
        base = wid * bpw

        def body(g, carry):
            start = base + g * ch
            pltpu.sync_copy(idx_hbm.at[pl.ds(start, ch)], idx_v)
            pltpu.async_copy(table_hbm.at[idx_v], rows_v, sem).wait()
            pltpu.sync_copy(rows_v, out_hbm.at[pl.ds(start, ch)])
            return carry

        lax.fori_loop(0, bpw // ch, body, 0)

    return k(table, idx)


def kernel(latents, prototypes):
    latents = latents.astype(jnp.float32)
    prototypes = prototypes.astype(jnp.float32)
    n = latents.shape[0]
    ln = jnp.sum(latents ** 2, axis=1, keepdims=True)
    pn = jnp.sum(prototypes ** 2, axis=1)[None, :]
    idx2d, loss2d = _tc_call(latents, ln, pn, prototypes)
    idx = idx2d.reshape(n)
    quantized = _sc_gather(prototypes, idx)
    return quantized, loss2d.reshape(())


# speedup vs baseline: 1.2731x; 1.2731x over previous
import functools

import jax
import jax.numpy as jnp
from jax import lax
from jax.experimental import pallas as pl
from jax.experimental.pallas import tpu as pltpu
from jax.experimental.pallas import tpu_sc as plsc

NUM_K = 8192
DIM = 256
ALPHA = 0.25
ENT_W = 0.01
BN = 256
CW = 512
LANES = 128


def _vq_tc_body(lat_ref, ln_ref, pn_ref, proto_ref, idx_ref, loss_ref,
                d_scr, e_scr, colacc, sums, *, nsteps, n_total):
    i = pl.program_id(0)

    @pl.when(i == 0)
    def _init():
        colacc[...] = jnp.zeros_like(colacc)
        sums[0, 0] = 0.0

    lt2 = lat_ref[...] + lat_ref[...]
    ln = ln_ref[...]
    nchunks = NUM_K // CW
    nh = CW // LANES
    lanef = lax.broadcasted_iota(jnp.int32, (1, LANES), 1).astype(jnp.float32)

    def body_a(c, carry):
        cmin, cidx = carry
        sl = pl.ds(c * CW, CW)
        ptc = proto_ref[sl, :]
        mmc = lax.dot_general(lt2, ptc, (((1,), (1,)), ((), ())),
                              preferred_element_type=jnp.float32)
        dchunk = (ln + pn_ref[:, sl]) - mmc
        d_scr[:, sl] = dchunk
        hmin = dchunk[:, 0:LANES]
        hcol = jnp.zeros((BN, LANES), jnp.float32)
        for h in range(1, nh):
            half = dchunk[:, h * LANES:(h + 1) * LANES]
            upd = half < hmin
            hmin = jnp.minimum(hmin, half)
            hcol = jnp.where(upd, float(h), hcol)
        updc = hmin < cmin
        cmin = jnp.minimum(cmin, hmin)
        cidx = jnp.where(updc, hcol + (c * nh).astype(jnp.float32), cidx)
        return cmin, cidx

    cmin0 = jnp.full((BN, LANES), jnp.inf, jnp.float32)
    cidx0 = jnp.zeros((BN, LANES), jnp.float32)
    cmin, cidx = lax.fori_loop(0, nchunks, body_a, (cmin0, cidx0),
                               unroll=2)

    minv = jnp.min(cmin, axis=1, keepdims=True)
    j_lane = cidx * float(LANES) + lanef
    j_sel = jnp.where(cmin == minv, j_lane, float(NUM_K))
    idxf = jnp.min(j_sel, axis=1, keepdims=True)
    idx_ref[...] = idxf.astype(jnp.int32)
    sums[0, 0] += jnp.sum(minv)

    def body_b(c, zacc):
        sl = pl.ds(c * CW, CW)
        e = jnp.exp(minv - d_scr[:, sl])
        e_scr[:, sl] = e
        zh = e[:, 0:LANES]
        for h in range(1, nh):
            zh = zh + e[:, h * LANES:(h + 1) * LANES]
        return zacc + zh

    zacc = lax.fori_loop(0, nchunks, body_b,
                         jnp.zeros((BN, LANES), jnp.float32), unroll=2)
    z = jnp.sum(zacc, axis=1, keepdims=True)
    zinv = 1.0 / z

    def body_c(c, carry):
        sl = pl.ds(c * CW, CW)
        scaled = e_scr[:, sl] * zinv
        colacc[:, sl] += jnp.sum(scaled, axis=0, keepdims=True)
        return carry

    lax.fori_loop(0, nchunks, body_c, 0, unroll=2)

    @pl.when(i == nsteps - 1)
    def _fin():
        s = colacc[...] * (1.0 / n_total) + 1e-6
        s = s * (1.0 / jnp.sum(s))
        ent = -jnp.sum(s * jnp.log(s))
        val = (sums[0, 0] * ((1.0 + ALPHA) / (n_total * DIM)) + ENT_W * ent)
        loss_ref[...] = jnp.reshape(val, (1, 1))


def _tc_call(latents, ln, pn, prototypes, interpret=False):
    n = latents.shape[0]
    nsteps = n // BN
    return pl.pallas_call(
        functools.partial(_vq_tc_body, nsteps=nsteps, n_total=n),
        grid=(nsteps,),
        in_specs=[
            pl.BlockSpec((BN, DIM), lambda i: (i, 0)),
            pl.BlockSpec((BN, 1), lambda i: (i, 0)),
            pl.BlockSpec((1, NUM_K), lambda i: (0, 0)),
            pl.BlockSpec((NUM_K, DIM), lambda i: (0, 0)),
        ],
        out_specs=[
            pl.BlockSpec((BN, 1), lambda i: (i, 0)),
            pl.BlockSpec((1, 1), lambda i: (0, 0)),
        ],
        out_shape=[
            jax.ShapeDtypeStruct((n, 1), jnp.int32),
            jax.ShapeDtypeStruct((1, 1), jnp.float32),
        ],
        scratch_shapes=[
            pltpu.VMEM((BN, NUM_K), jnp.float32),
            pltpu.VMEM((BN, NUM_K), jnp.float32),
            pltpu.VMEM((1, NUM_K), jnp.float32),
            pltpu.SMEM((1, 1), jnp.float32),
        ],
        interpret=interpret,
    )(latents, ln, pn, prototypes)


def _sc_gather(table, idx):
    n = idx.shape[0]
    info = plsc.get_sparse_core_info()
    nw = info.num_cores * info.num_subcores
    bpw = n // nw
    ch = 128
    mesh = plsc.VectorSubcoreMesh(core_axis_name="c", subcore_axis_name="s")

    @functools.partial(
        pl.kernel, mesh=mesh,
        out_type=jax.ShapeDtypeStruct((n, DIM), jnp.float32),
        scratch_types=[
            pltpu.VMEM((ch,), jnp.int32),
            pltpu.VMEM((ch, DIM), jnp.float32),
            pltpu.SemaphoreType.DMA,
        ],
    )
    def k(table_hbm, idx_hbm, out_hbm, idx_v, rows_v, sem):
        wid = lax.axis_index("s") * info.num_cores + lax.axis_index("c")
        base = wid * bpw

        def body(g, carry):
            start = base + g * ch
            pltpu.sync_copy(idx_hbm.at[pl.ds(start, ch)], idx_v)
            pltpu.async_copy(table_hbm.at[idx_v], rows_v, sem).wait()
            pltpu.sync_copy(rows_v, out_hbm.at[pl.ds(start, ch)])
            return carry

        lax.fori_loop(0, bpw // ch, body, 0)

    return k(table, idx)


def kernel(latents, prototypes):
    latents = latents.astype(jnp.float32)
    prototypes = prototypes.astype(jnp.float32)
    n = latents.shape[0]
    ln = jnp.sum(latents ** 2, axis=1, keepdims=True)
    pn = jnp.sum(prototypes ** 2, axis=1)[None, :]
    idx2d, loss2d = _tc_call(latents, ln, pn, prototypes)
    idx = idx2d.reshape(n)
    quantized = _sc_gather(prototypes, idx)
    return quantized, loss2d.reshape(())


# speedup vs baseline: 1.7100x; 1.3432x over previous
import functools

import jax
import jax.numpy as jnp
from jax import lax
from jax.experimental import pallas as pl
from jax.experimental.pallas import tpu as pltpu
from jax.experimental.pallas import tpu_sc as plsc

NUM_K = 8192
DIM = 256
ALPHA = 0.25
ENT_W = 0.01
BN = 256


def _vq_tc_body(lat_ref, ln_ref, pn_ref, proto_ref, idx_ref, loss_ref,
                colacc, sums, *, nsteps, n_total):
    i = pl.program_id(0)

    @pl.when(i == 0)
    def _init():
        colacc[...] = jnp.zeros_like(colacc)
        sums[0, 0] = 0.0

    lt = lat_ref[...]
    pt = proto_ref[...]
    mm = lax.dot_general(lt, pt, (((1,), (1,)), ((), ())),
                         preferred_element_type=jnp.float32)
    d = (ln_ref[...] + pn_ref[...]) - 2.0 * mm
    minv = jnp.min(d, axis=1, keepdims=True)
    jidxf = lax.broadcasted_iota(jnp.int32, (1, NUM_K), 1).astype(jnp.float32)
    idxf = jnp.min(jnp.where(d == minv, jidxf, float(NUM_K)),
                   axis=1, keepdims=True)
    idx_ref[...] = idxf.astype(jnp.int32)

    e = jnp.exp(minv - d)
    z = jnp.sum(e, axis=1, keepdims=True)
    colacc[...] += jnp.sum(e * (1.0 / z), axis=0, keepdims=True)
    sums[0, 0] += jnp.sum(minv)

    @pl.when(i == nsteps - 1)
    def _fin():
        s = colacc[...] * (1.0 / n_total) + 1e-6
        s = s * (1.0 / jnp.sum(s))
        ent = -jnp.sum(s * jnp.log(s))
        val = (sums[0, 0] * ((1.0 + ALPHA) / (n_total * DIM)) + ENT_W * ent)
        loss_ref[...] = jnp.reshape(val, (1, 1))


def _tc_call(latents, ln, pn, prototypes, interpret=False):
    n = latents.shape[0]
    nsteps = n // BN
    return pl.pallas_call(
        functools.partial(_vq_tc_body, nsteps=nsteps, n_total=n),
        grid=(nsteps,),
        in_specs=[
            pl.BlockSpec((BN, DIM), lambda i: (i, 0)),
            pl.BlockSpec((BN, 1), lambda i: (i, 0)),
            pl.BlockSpec((1, NUM_K), lambda i: (0, 0)),
            pl.BlockSpec((NUM_K, DIM), lambda i: (0, 0)),
        ],
        out_specs=[
            pl.BlockSpec((BN, 1), lambda i: (i, 0)),
            pl.BlockSpec((1, 1), lambda i: (0, 0)),
        ],
        out_shape=[
            jax.ShapeDtypeStruct((n, 1), jnp.int32),
            jax.ShapeDtypeStruct((1, 1), jnp.float32),
        ],
        scratch_shapes=[
            pltpu.VMEM((1, NUM_K), jnp.float32),
            pltpu.SMEM((1, 1), jnp.float32),
        ],
        interpret=interpret,
    )(latents, ln, pn, prototypes)


def _sc_gather(table, idx):
    n = idx.shape[0]
    info = plsc.get_sparse_core_info()
    nw = info.num_cores * info.num_subcores
    bpw = n // nw
    ch = 128
    mesh = plsc.VectorSubcoreMesh(core_axis_name="c", subcore_axis_name="s")

    @functools.partial(
        pl.kernel, mesh=mesh,
        out_type=jax.ShapeDtypeStruct((n, DIM), jnp.float32),
        scratch_types=[
            pltpu.VMEM((ch,), jnp.int32),
            pltpu.VMEM((ch, DIM), jnp.float32),
            pltpu.SemaphoreType.DMA,
        ],
    )
    def k(table_hbm, idx_hbm, out_hbm, idx_v, rows_v, sem):
        wid = lax.axis_index("s") * info.num_cores + lax.axis_index("c")
        base = wid * bpw

        def body(g, carry):
            start = base + g * ch
            pltpu.sync_copy(idx_hbm.at[pl.ds(start, ch)], idx_v)
            pltpu.async_copy(table_hbm.at[idx_v], rows_v, sem).wait()
            pltpu.sync_copy(rows_v, out_hbm.at[pl.ds(start, ch)])
            return carry

        lax.fori_loop(0, bpw // ch, body, 0)

    return k(table, idx)


def kernel(latents, prototypes):
    latents = latents.astype(jnp.float32)
    prototypes = prototypes.astype(jnp.float32)
    n = latents.shape[0]
    ln = jnp.sum(latents ** 2, axis=1, keepdims=True)
    pn = jnp.sum(prototypes ** 2, axis=1)[None, :]
    idx2d, loss2d = _tc_call(latents, ln, pn, prototypes)
    idx = idx2d.reshape(n)
    quantized = _sc_gather(prototypes, idx)
    return quantized, loss2d.reshape(())
